# Initial kernel scaffold; baseline (speedup 1.0000x reference)
import jax
import jax.numpy as jnp
from jax.experimental import pallas as pl


def kernel(x, edge_index, u, batch, W1, b1, W2, b2, W3, b3):
    raise NotImplementedError("write your pallas kernel here")


# speedup vs baseline: 11.8596x; 11.8596x over previous
import jax
import jax.numpy as jnp
from jax.experimental import pallas as pl

N = 10000
D = 128
G = 64
ING = 128
H = 256
OUT = 128


def _fused_kernel(x_ref, u_ref, batch_ref, w1_ref, b1_ref, w2_ref, b2_ref,
                  w3_ref, b3_ref, out_ref):
    batch = batch_ref[0, :]
    seg_ids = jax.lax.broadcasted_iota(jnp.int32, (G, N), 0)
    onehot = (batch[None, :] == seg_ids).astype(jnp.float32)
    x = x_ref[...]
    seg_sum = jnp.dot(onehot, x, preferred_element_type=jnp.float32)
    cnt = jnp.sum(onehot, axis=1, keepdims=True)
    seg_mean = seg_sum / jnp.maximum(cnt, 1.0)
    cat = jnp.concatenate([u_ref[...], seg_mean], axis=1)
    h = jnp.dot(cat, w1_ref[...], preferred_element_type=jnp.float32)
    h = jnp.maximum(h + b1_ref[...], 0.0)
    h = jnp.dot(h, w2_ref[...], preferred_element_type=jnp.float32)
    h = jnp.maximum(h + b2_ref[...], 0.0)
    h = jnp.dot(h, w3_ref[...], preferred_element_type=jnp.float32)
    out_ref[...] = h + b3_ref[...]


def kernel(x, edge_index, u, batch, W1, b1, W2, b2, W3, b3):
    del edge_index
    batch2d = batch.reshape(1, N)
    args = (x, u, batch2d, W1.T, b1.reshape(1, H), W2.T, b2.reshape(1, H),
            W3.T, b3.reshape(1, OUT))
    return pl.pallas_call(
        _fused_kernel,
        out_shape=jax.ShapeDtypeStruct((G, OUT), jnp.float32),
    )(*args)
